# Initial kernel scaffold; baseline (speedup 1.0000x reference)
import jax
import jax.numpy as jnp
from jax.experimental import pallas as pl


def kernel(x, flow, sample_grid):
    raise NotImplementedError("write your pallas kernel here")


# speedup vs baseline: 1.0438x; 1.0438x over previous
import functools

import jax
import jax.numpy as jnp
from jax import lax
from jax.experimental import pallas as pl
from jax.experimental.pallas import tpu as pltpu
from jax.experimental.pallas import tpu_sc as plsc


def _index_body(W, H, sgx_ref, sgy_ref, flow_ref, out_ref):
    fx = flow_ref[0, 0]
    fy = flow_ref[0, 1]
    fxt = fx.T
    fyt = fy.T
    gx = sgx_ref[0] + fxt
    gy = sgy_ref[0] + fyt
    ix = jnp.clip(jnp.round((gx + 1.0) * 0.5 * (W - 1)), 0, W - 1).astype(jnp.int32)
    iy = jnp.clip(jnp.round((gy + 1.0) * 0.5 * (H - 1)), 0, H - 1).astype(jnp.int32)
    out_ref[0] = iy * W + ix


def _make_index_kernel(B, H, W, hb):
    grid = (B, H // hb)
    return pl.pallas_call(
        functools.partial(_index_body, W, H),
        grid=grid,
        in_specs=[
            pl.BlockSpec((1, hb, W), lambda b, i: (b, i, 0)),
            pl.BlockSpec((1, hb, W), lambda b, i: (b, i, 0)),
            pl.BlockSpec((1, 2, W, hb), lambda b, i: (b, 0, 0, i)),
        ],
        out_specs=pl.BlockSpec((1, hb, W), lambda b, i: (b, i, 0)),
        out_shape=jax.ShapeDtypeStruct((B, H, W), jnp.int32),
    )


def _make_gather_kernel(B, C, HW, sub=2048):
    NW = 32
    chunk = HW // NW
    nsub = chunk // sub
    mesh = plsc.VectorSubcoreMesh(core_axis_name="c", subcore_axis_name="s")

    @functools.partial(
        pl.kernel,
        mesh=mesh,
        compiler_params=pltpu.CompilerParams(use_tc_tiling_on_sc=False),
        out_type=jax.ShapeDtypeStruct((B, HW, C), jnp.float32),
        scratch_types=[
            pltpu.VMEM((chunk,), jnp.int32),
            pltpu.VMEM((sub, C), jnp.float32),
            pltpu.SemaphoreType.DMA,
        ],
    )
    def gather(x_hbm, idx_hbm, out_hbm, idx_v, data_v, sem):
        wid = lax.axis_index("s") * 2 + lax.axis_index("c")
        base = wid * chunk

        def b_loop(b, carry):
            pltpu.sync_copy(idx_hbm.at[b, pl.ds(base, chunk)], idx_v)

            def s_loop(s, carry2):
                pltpu.async_copy(
                    x_hbm.at[b].at[idx_v.at[pl.ds(s * sub, sub)]], data_v, sem
                ).wait()
                pltpu.sync_copy(data_v, out_hbm.at[b, pl.ds(base + s * sub, sub), :])
                return carry2

            return lax.fori_loop(0, nsub, s_loop, carry)

        lax.fori_loop(0, B, b_loop, 0)

    return gather


def kernel(x, flow, sample_grid):
    B, C, H, W = x.shape
    sgx = sample_grid[..., 0]
    sgy = sample_grid[..., 1]
    idx = _make_index_kernel(B, H, W, 128)(sgx, sgy, flow)
    x_nhwc = jnp.transpose(x.reshape(B, C, H * W), (0, 2, 1))
    out_nhwc = _make_gather_kernel(B, C, H * W)(x_nhwc, idx.reshape(B, H * W))
    return jnp.transpose(out_nhwc, (0, 2, 1)).reshape(B, C, H, W)
